# Initial kernel scaffold; baseline (speedup 1.0000x reference)
import jax
import jax.numpy as jnp
from jax.experimental import pallas as pl


def kernel(feats, coords, W1, b1, g1, beta1, W2, b2, g2, beta2, W3, b3):
    raise NotImplementedError("write your pallas kernel here")


# speedup vs baseline: 3.2838x; 3.2838x over previous
import functools

import jax
import jax.numpy as jnp
from jax.experimental import pallas as pl

N = 8192
K = 8
D = 66
ROWS = 128
NOFF = 27


def _knn_kernel(pt_ref, ci_ref, idx_ref, nbr_ref, msk_ref):
    i = pl.program_id(0)
    p_all = pt_ref[0:3, :]
    p_blk = pt_ref[0:3, pl.ds(i * ROWS, ROWS)]
    x2a = p_all[0:1, :] * p_all[0:1, :]
    y2a = p_all[1:2, :] * p_all[1:2, :]
    z2a = p_all[2:3, :] * p_all[2:3, :]
    sq_all = (x2a + z2a) + y2a
    x2b = p_blk[0:1, :] * p_blk[0:1, :]
    y2b = p_blk[1:2, :] * p_blk[1:2, :]
    z2b = p_blk[2:3, :] * p_blk[2:3, :]
    sq_blk = (x2b + z2b) + y2b
    dot = jax.lax.dot_general(
        p_blk, p_all, (((0,), (0,)), ((), ())),
        preferred_element_type=jnp.float32)
    d2 = (jnp.transpose(sq_blk) + sq_all) - 2.0 * dot
    jidx = jax.lax.broadcasted_iota(jnp.int32, (ROWS, N), 1)
    big = jnp.int32(2**30)

    cols = [jax.lax.broadcasted_iota(jnp.int32, (ROWS, 1), 0) + i * ROWS]
    d2w = d2
    for _ in range(K):
        m = jnp.min(d2w, axis=1, keepdims=True)
        am = jnp.min(jnp.where(d2w == m, jidx, big), axis=1, keepdims=True)
        cols.append(am)
        d2w = jnp.where(jidx == am, jnp.float32(jnp.inf), d2w)
    cols.append(jnp.zeros((ROWS, 16 - K - 1), jnp.int32))
    idx_ref[...] = jnp.concatenate(cols, axis=1)

    bj = ci_ref[0:1, :]
    xj = ci_ref[1:2, :]
    yj = ci_ref[2:3, :]
    zj = ci_ref[3:4, :]
    bi = jnp.transpose(ci_ref[0:1, pl.ds(i * ROWS, ROWS)])
    xi = jnp.transpose(ci_ref[1:2, pl.ds(i * ROWS, ROWS)])
    yi = jnp.transpose(ci_ref[2:3, pl.ds(i * ROWS, ROWS)])
    zi = jnp.transpose(ci_ref[3:4, pl.ds(i * ROWS, ROWS)])
    dx = xj - xi
    dy = yj - yi
    dz = zj - zi
    one = jnp.int32(1)
    valid = ((bj == bi) & (jnp.abs(dx) <= one) & (jnp.abs(dy) <= one)
             & (jnp.abs(dz) <= one))
    code = jnp.where(valid, (dx + 1) * 9 + (dy + 1) * 3 + (dz + 1),
                     jnp.int32(-1))
    ncols = []
    for k in range(NOFF):
        ncols.append(jnp.min(jnp.where(code == k, jidx, big), axis=1,
                             keepdims=True))
    ncols.append(jnp.full((ROWS, 32 - NOFF), big, jnp.int32))
    nbr = jnp.concatenate(ncols, axis=1)
    msk = nbr < jnp.int32(N)
    nbr_ref[...] = jnp.where(msk, nbr, 0)
    msk_ref[...] = msk.astype(jnp.float32)


def _knn_pallas(coords):
    b = coords[:, 0].astype(jnp.float32)
    xyz = coords[:, 1:].astype(jnp.float32)
    p = xyz + b[:, None] * 1e4
    pt = jnp.zeros((8, N), jnp.float32).at[0:3, :].set(p.T)
    ci = jnp.zeros((8, N), jnp.int32)
    ci = ci.at[0, :].set(coords[:, 0])
    ci = ci.at[1:4, :].set(coords[:, 1:].T + 1)
    return pl.pallas_call(
        _knn_kernel,
        grid=(N // ROWS,),
        in_specs=[pl.BlockSpec((8, N), lambda i: (0, 0)),
                  pl.BlockSpec((8, N), lambda i: (0, 0))],
        out_specs=[pl.BlockSpec((ROWS, 16), lambda i: (i, 0)),
                   pl.BlockSpec((ROWS, 32), lambda i: (i, 0)),
                   pl.BlockSpec((ROWS, 32), lambda i: (i, 0))],
        out_shape=[jax.ShapeDtypeStruct((N, 16), jnp.int32),
                   jax.ShapeDtypeStruct((N, 32), jnp.int32),
                   jax.ShapeDtypeStruct((N, 32), jnp.float32)],
    )(pt, ci)


def _bn(x, g, b, eps=1e-5):
    m = jnp.mean(x, axis=0)
    v = jnp.var(x, axis=0)
    return (x - m) / jnp.sqrt(v + eps) * g + b


def kernel(feats, coords, W1, b1, g1, beta1, W2, b2, g2, beta2, W3, b3):
    idx9, nbr, msk = _knn_pallas(coords)
    g9 = feats[idx9[:, :K + 1].reshape(-1)].reshape(N, (K + 1) * feats.shape[1])
    h = jax.nn.relu(_bn(g9 @ W1 + b1, g1, beta1))
    gh = h[nbr[:, :NOFF].reshape(-1)].reshape(N, NOFF, h.shape[1])
    gh = gh * msk[:, :NOFF, None]
    conv = jnp.einsum('nkc,kcd->nd', gh, W2) + b2
    h2 = jax.nn.relu(_bn(conv, g2, beta2))
    return h2 @ W3 + b3
